# baseline (speedup 1.0000x reference)
import functools

import jax
import jax.numpy as jnp
from jax import lax
from jax.experimental import pallas as pl
from jax.experimental.pallas import tpu as pltpu
from jax.experimental.pallas import tpu_sc as plsc

N = 10000
E = 320000
D = 128
H = 128
G = 128

_info = plsc.get_sparse_core_info()
NC = _info.num_cores
NS = _info.num_subcores
NW = NC * NS
NG = E // G
GPT_LO = NG // NW
N_HI = NG % NW
ROWS_PT = 632
ACC_ROWS = ROWS_PT * NS


def _bn_body(x_ref, w_ref, b_ref, xn_ref):
    x = x_ref[...]
    mu = jnp.mean(x, axis=0, keepdims=True)
    xc = x - mu
    var = jnp.mean(xc * xc, axis=0, keepdims=True)
    xn_ref[...] = xc * lax.rsqrt(var + 1e-5) * w_ref[...] + b_ref[...]


_bn_call = pl.pallas_call(
    _bn_body,
    out_shape=jax.ShapeDtypeStruct((N, D), jnp.float32),
)


_mesh = plsc.VectorSubcoreMesh(core_axis_name="c", subcore_axis_name="s")


CW = 16


@functools.partial(
    pl.kernel,
    out_type=(jax.ShapeDtypeStruct((NC, ACC_ROWS, D), jnp.float32),
              jax.ShapeDtypeStruct((NC, ACC_ROWS, CW), jnp.float32)),
    mesh=_mesh,
    scratch_types=[
        pltpu.VMEM((4, G), jnp.int32),
        pltpu.VMEM((4, G), jnp.int32),
        pltpu.VMEM((2, G, D), jnp.float32),
        pltpu.VMEM((G, CW), jnp.float32),
        pltpu.VMEM_SHARED((ACC_ROWS, D), jnp.float32),
        pltpu.VMEM_SHARED((ACC_ROWS, CW), jnp.float32),
        pltpu.SemaphoreType.DMA,
        pltpu.SemaphoreType.DMA,
        pltpu.SemaphoreType.DMA,
        pltpu.SemaphoreType.DMA,
        pltpu.SemaphoreType.DMA,
        pltpu.SemaphoreType.DMA,
        pltpu.SemaphoreType.DMA,
        pltpu.SemaphoreType.DMA,
        pltpu.SemaphoreType.DMA,
        pltpu.SemaphoreType.DMA,
    ],
    compiler_params=pltpu.CompilerParams(use_tc_tiling_on_sc=False),
)
def _sc_scatter(xn_hbm, edge_hbm, zeros_hbm, zeros16_hbm, ones_hbm,
                out_hbm, cnt_hbm,
                src_v, dst_v, rows_v, ones_v, acc_sh, cnt_sh,
                isem0, isem1, isem2, isem3,
                grsem0, grsem1, fsem0, fsem1, csem0, csem1):
    c = lax.axis_index("c")
    s = lax.axis_index("s")
    wid = s * NC + c
    n_g = GPT_LO + jnp.where(wid < N_HI, 1, 0)
    base = wid * GPT_LO + jnp.minimum(wid, N_HI)
    isems = (isem0, isem1, isem2, isem3)
    grsems = (grsem0, grsem1)
    fsems = (fsem0, fsem1)
    csems = (csem0, csem1)

    def idx_fetch(g, r):
        off = (base + g) * G
        pltpu.async_copy(edge_hbm.at[0].at[pl.ds(off, G)], src_v.at[r],
                         isems[r])
        pltpu.async_copy(edge_hbm.at[1].at[pl.ds(off, G)], dst_v.at[r],
                         isems[r])

    def idx_wait(g, r):
        off = (base + g) * G
        pltpu.make_async_copy(edge_hbm.at[0].at[pl.ds(off, G)], src_v.at[r],
                              isems[r]).wait()
        pltpu.make_async_copy(edge_hbm.at[1].at[pl.ds(off, G)], dst_v.at[r],
                              isems[r]).wait()

    def feat_wait(p):
        pltpu.make_async_copy(rows_v.at[p], acc_sh.at[dst_v.at[0]],
                              fsems[p]).wait()

    def cnt_wait(p):
        pltpu.make_async_copy(ones_v, cnt_sh.at[dst_v.at[0]],
                              csems[p]).wait()

    idx_fetch(0, 0)
    idx_fetch(1, 1)
    idx_fetch(2, 2)
    idx_fetch(3, 3)
    pltpu.sync_copy(ones_hbm, ones_v)
    pltpu.sync_copy(zeros_hbm.at[pl.ds(s * ROWS_PT, ROWS_PT)],
                    acc_sh.at[pl.ds(s * ROWS_PT, ROWS_PT)])
    pltpu.sync_copy(zeros16_hbm.at[pl.ds(s * ROWS_PT, ROWS_PT)],
                    cnt_sh.at[pl.ds(s * ROWS_PT, ROWS_PT)])
    plsc.subcore_barrier()

    idx_wait(0, 0)
    pltpu.async_copy(xn_hbm.at[src_v.at[0]], rows_v.at[0], grsem0)

    def step(g, k):
        p = k % 2
        q = 1 - p
        pltpu.make_async_copy(xn_hbm.at[src_v.at[k]], rows_v.at[p],
                              grsems[p]).wait()

        @pl.when(g >= 1)
        def _():
            feat_wait(q)
            cnt_wait(q)

        @pl.when(g + 1 < n_g)
        def _():
            idx_wait(g + 1, (k + 1) % 4)
            pltpu.async_copy(xn_hbm.at[src_v.at[(k + 1) % 4]], rows_v.at[q],
                             grsems[q])

        pltpu.async_copy(rows_v.at[p], acc_sh.at[dst_v.at[k]], fsems[p],
                         add=True)
        pltpu.async_copy(ones_v, cnt_sh.at[dst_v.at[k]], csems[p],
                         add=True)

        @pl.when((g + 3 < n_g) & (g >= 1))
        def _():
            idx_fetch(g + 3, (k + 3) % 4)

    def body(ii, carry):
        g0 = ii * 4
        step(g0, 0)
        for k in (1, 2, 3):
            @pl.when(g0 + k < n_g)
            def _(k=k):
                step(g0 + k, k)

        return carry

    lax.fori_loop(0, (n_g + 3) // 4, body, 0)
    last_p = (n_g - 1) % 2

    @pl.when(last_p == 0)
    def _():
        feat_wait(0)
        cnt_wait(0)

    @pl.when(last_p == 1)
    def _():
        feat_wait(1)
        cnt_wait(1)

    plsc.subcore_barrier()
    pltpu.sync_copy(acc_sh.at[pl.ds(s * ROWS_PT, ROWS_PT)],
                    out_hbm.at[c].at[pl.ds(s * ROWS_PT, ROWS_PT)])
    pltpu.sync_copy(cnt_sh.at[pl.ds(s * ROWS_PT, ROWS_PT)],
                    cnt_hbm.at[c].at[pl.ds(s * ROWS_PT, ROWS_PT)])


_DN = (((1,), (1,)), ((), ()))


def _head_body(a_ref, cnt_ref, xn_ref, wl_ref, bl_ref, wr_ref,
               wc1_ref, bc1_ref, wc2_ref, bc2_ref, out_ref):
    a = a_ref[0] + a_ref[1]
    cnt = (cnt_ref[0] + cnt_ref[1])[:, :1]
    agg = a / jnp.maximum(cnt, 1.0)
    xn = xn_ref[...]
    hp = lax.Precision.DEFAULT
    h = (lax.dot_general(agg, wl_ref[...], _DN, precision=hp)
         + lax.dot_general(xn, wr_ref[...], _DN, precision=hp)
         + bl_ref[...])
    h = jnp.maximum(h, 0.0)
    h1 = jnp.maximum(
        lax.dot_general(h, wc1_ref[...], _DN, precision=hp) + bc1_ref[...],
        0.0)
    out_ref[...] = (lax.dot_general(h1, wc2_ref[...], _DN, precision=hp)
                    + bc2_ref[...])


_BN = 2000

_head_call = pl.pallas_call(
    _head_body,
    grid=(N // _BN,),
    in_specs=[
        pl.BlockSpec((2, _BN, D), lambda i: (0, i, 0)),
        pl.BlockSpec((2, _BN, CW), lambda i: (0, i, 0)),
        pl.BlockSpec((_BN, D), lambda i: (i, 0)),
        pl.BlockSpec((H, D), lambda i: (0, 0)),
        pl.BlockSpec((1, H), lambda i: (0, 0)),
        pl.BlockSpec((H, D), lambda i: (0, 0)),
        pl.BlockSpec((16, H), lambda i: (0, 0)),
        pl.BlockSpec((1, 16), lambda i: (0, 0)),
        pl.BlockSpec((2, 16), lambda i: (0, 0)),
        pl.BlockSpec((1, 2), lambda i: (0, 0)),
    ],
    out_specs=pl.BlockSpec((_BN, 2), lambda i: (i, 0)),
    out_shape=jax.ShapeDtypeStruct((N, 2), jnp.float32),
)


def kernel(x, edge_index, edge_weight, edge_features, adj, T,
           bn_weight, bn_bias, W_l, b_l, W_r, Wc1, bc1, Wc2, bc2):
    xn = _bn_call(x, bn_weight.reshape(1, D), bn_bias.reshape(1, D))
    zeros = jnp.zeros((ACC_ROWS, D), jnp.float32)
    zeros16 = jnp.zeros((ACC_ROWS, CW), jnp.float32)
    ones = jnp.ones((G, CW), jnp.float32)
    acc, cnts = _sc_scatter(xn, edge_index, zeros, zeros16, ones)
    out = _head_call(acc, cnts, xn, W_l, b_l.reshape(1, H), W_r,
                     Wc1, bc1.reshape(1, 16), Wc2, bc2.reshape(1, 2))
    return out
